# baseline (speedup 1.0000x reference)
import functools

import jax
import jax.numpy as jnp
from jax import lax
from jax.experimental import pallas as pl
from jax.experimental.pallas import tpu as pltpu
from jax.experimental.pallas import tpu_sc as plsc

NUM_EMBEDDINGS = 100000
EMBED_DIM = 128
BATCH = 4096
HIST_LEN = 50

_HP = 56
_NC = 2
_NS = 16
_NW = _NC * _NS
_BPW = BATCH // _NW

_NBUF = 4
_G = 2
_T = _BPW // _NBUF


def _make_gather():
    mesh = plsc.VectorSubcoreMesh(core_axis_name="c", subcore_axis_name="s")

    @functools.partial(
        pl.kernel,
        mesh=mesh,
        out_type=jax.ShapeDtypeStruct((BATCH, HIST_LEN, EMBED_DIM),
                                      jnp.float32),
        scratch_types=[
            pltpu.VMEM((_BPW * _HP,), jnp.int32),
            pltpu.VMEM((_NBUF, _HP, EMBED_DIM), jnp.float32),
        ]
        + [pltpu.SemaphoreType.DMA] * _NBUF,
    )
    def gather_kernel(idx_hbm, table_hbm, out_hbm, idx_v, rows_v,
                      sem0, sem1, sem2, sem3):
        sems = [sem0, sem1, sem2, sem3]
        wid = lax.axis_index("s") * _NC + lax.axis_index("c")
        base_b = wid * _BPW
        pltpu.sync_copy(idx_hbm.at[pl.ds(base_b * _HP, _BPW * _HP)], idx_v)

        def g_start(q, b):
            pltpu.async_copy(
                table_hbm.at[idx_v.at[pl.ds(q * _HP, _HP)]], rows_v.at[b],
                sems[b])

        def g_wait(q, b):
            pltpu.make_async_copy(
                table_hbm.at[idx_v.at[pl.ds(q * _HP, _HP)]], rows_v.at[b],
                sems[b]).wait()

        def s_start(q, b):
            pltpu.async_copy(rows_v.at[b, pl.ds(0, HIST_LEN)],
                             out_hbm.at[base_b + q], sems[b])

        def s_wait(q, b):
            pltpu.make_async_copy(rows_v.at[b, pl.ds(0, HIST_LEN)],
                                  out_hbm.at[base_b + q], sems[b]).wait()

        for c in range(_G):
            g_start(c, c % _NBUF)

        def slot(q, b, c_static=None):
            g_wait(q, b)
            s_start(q, b)
            c = q + _G if c_static is None else c_static
            bc = (b + _G) % _NBUF
            if c_static is None or c_static >= _NBUF:
                s_wait(c - _NBUF, bc)
            g_start(c, bc)

        for b in range(_NBUF):
            slot(b, b, c_static=b + _G)

        def outer(t, carry):
            for b in range(_NBUF):
                slot(t * _NBUF + b, b)
            return carry

        lax.fori_loop(1, _T - 1, outer, 0)

        for b in range(_NBUF):
            q = (_T - 1) * _NBUF + b
            g_wait(q, b)
            s_start(q, b)
            c = q + _G
            if c < _BPW:
                s_wait(c - _NBUF, c % _NBUF)
                g_start(c, c % _NBUF)

        for q in range(_BPW - _NBUF, _BPW):
            s_wait(q, q % _NBUF)

    return gather_kernel


_gather = _make_gather()


def kernel(input_, weight):
    idx = jnp.pad(input_.astype(jnp.int32), ((0, 0), (0, _HP - HIST_LEN)))
    return _gather(idx.reshape(-1), weight)
